# baseline (speedup 1.0000x reference)
import functools
from math import sqrt

import jax
import jax.numpy as jnp
from jax import lax
from jax.experimental import pallas as pl
from jax.experimental.pallas import tpu as pltpu
from jax.experimental.pallas import tpu_sc as plsc

_VOCAB = 1000000
_D = 64
_ROWS = 4096
_COLS = 200
_B = _ROWS * _COLS
_NC = 2
_NS = 16
_NW = _NC * _NS
_PER_W = _B // _NW
_CHUNK = 128
_NCHUNK = _PER_W // _CHUNK
_NBUF = 8
_GDEPTH = 4
_SCALE = sqrt(_D)
_TW = 4096
_TMAIN = (_VOCAB // (2 * _TW)) * (2 * _TW)
_TGRID = _TMAIN // (2 * _TW) + 1
_TTAIL = (_VOCAB - _TMAIN) // 2
_TPACK = _TGRID * _TW


def _eye(scale):
    r = lax.broadcasted_iota(jnp.int32, (_D, _D), 0)
    c = lax.broadcasted_iota(jnp.int32, (_D, _D), 1)
    return jnp.where(r == c, jnp.float32(scale), jnp.float32(0.0))


def _sel(shift, scale):
    r = lax.broadcasted_iota(jnp.int32, (_D, 128), 0)
    c = lax.broadcasted_iota(jnp.int32, (_D, 128), 1)
    return jnp.where(c == r + shift, jnp.float32(scale), jnp.float32(0.0))


def _tr(a, eye):
    return lax.dot_general(
        a, eye, (((0,), (0,)), ((), ())), preferred_element_type=jnp.float32
    )


def _pack_table_body(in0_ref, in1_ref, tail_ref, out_ref):
    i = pl.program_id(0)

    @pl.when(i < _TGRID - 1)
    def _():
        out_ref[...] = _tr(in0_ref[...], _sel(0, 1.0)) + _tr(
            in1_ref[...], _sel(_D, 1.0)
        )

    @pl.when(i == _TGRID - 1)
    def _():
        eye = _eye(1.0)
        out_ref[0:_TTAIL, 0:_D] = _tr(tail_ref[:, 0:_TTAIL], eye)
        out_ref[0:_TTAIL, _D:128] = _tr(tail_ref[:, _TTAIL : 2 * _TTAIL], eye)


@jax.jit
def _pack_table(w_t):
    last = _TGRID - 2

    return pl.pallas_call(
        _pack_table_body,
        grid=(_TGRID,),
        in_specs=[
            pl.BlockSpec((_D, _TW), lambda i: (0, 2 * jnp.minimum(i, last))),
            pl.BlockSpec((_D, _TW), lambda i: (0, 2 * jnp.minimum(i, last) + 1)),
            pl.BlockSpec((_D, 2 * _TTAIL), lambda i: (0, 0)),
        ],
        out_specs=pl.BlockSpec((_TW, 128), lambda i: (i, 0)),
        out_shape=jax.ShapeDtypeStruct((_TPACK, 128), jnp.float32),
    )(w_t, w_t, lax.slice(w_t, (0, _TMAIN), (_D, _VOCAB)))


def _gather_body(x_hbm, tab_hbm, out_hbm, idx_v, rows_v, gsem, wsem):
    cid = lax.axis_index("c")
    sid = lax.axis_index("s")
    wid = sid * _NC + cid
    pltpu.sync_copy(x_hbm.at[wid], idx_v)

    @pl.loop(0, _NCHUNK)
    def _remap(k):
        for v in range(_CHUNK // 16):
            sl = pl.ds(v * 16, 16)
            i = idx_v[k, sl]
            t = lax.bitwise_and(i, 2 * _TW - 1)
            main = (
                (i - t)
                + lax.shift_left(lax.bitwise_and(t, _TW - 1), 1)
                + lax.shift_right_logical(t, _TW.bit_length() - 1)
            )
            u = i - _TMAIN
            h = lax.div(u, _TTAIL)
            tail = _TMAIN + 2 * (u - _TTAIL * h) + h
            idx_v[k, sl] = jnp.where(i < _TMAIN, main, tail)

    def fire_gather(k, b):
        pltpu.async_copy(tab_hbm.at[idx_v.at[k]], rows_v.at[b], gsem)

    def drain_gather(b):
        pltpu.make_async_copy(tab_hbm.at[idx_v.at[0]], rows_v.at[b], gsem).wait()

    def fire_write(k, b):
        pltpu.async_copy(rows_v.at[b], out_hbm.at[wid, k], wsem)

    def drain_write(b):
        pltpu.make_async_copy(rows_v.at[b], out_hbm.at[wid, 0], wsem).wait()

    for b in range(_GDEPTH):
        fire_gather(b, b)

    @pl.loop(0, _NCHUNK - _NBUF, step=_NBUF)
    def _group(j):
        for b in range(_NBUF):
            k = j + b
            drain_gather(b)
            fire_write(k, b)
            b4 = (b + _GDEPTH) % _NBUF
            if b >= _GDEPTH:
                drain_write(b4)
            else:

                @pl.when(j > 0)
                def _():
                    drain_write(b4)

            fire_gather(k + _GDEPTH, b4)

    for b in range(_NBUF):
        k = _NCHUNK - _NBUF + b
        drain_gather(b)
        fire_write(k, b)
        if b < _GDEPTH:
            drain_write((b + _GDEPTH) % _NBUF)
            fire_gather(k + _GDEPTH, (b + _GDEPTH) % _NBUF)
    for b in range(_NBUF):
        drain_write(b)


@jax.jit
def _gather(x_flat, tab):
    k = pl.kernel(
        _gather_body,
        out_type=jax.ShapeDtypeStruct((_NW, _NCHUNK, _CHUNK, _D), jnp.float32),
        mesh=plsc.VectorSubcoreMesh(core_axis_name="c", subcore_axis_name="s"),
        scratch_types=[
            pltpu.VMEM((_NCHUNK, _CHUNK), jnp.int32),
            pltpu.VMEM((_NBUF, _CHUNK, _D), jnp.float32),
            pltpu.SemaphoreType.DMA,
            pltpu.SemaphoreType.DMA,
        ],
        compiler_params=pltpu.CompilerParams(use_tc_tiling_on_sc=False),
    )
    return k(x_flat, tab)


_FCB = 2


def _finish_body(in_ref, out_ref):
    half = _ROWS // 2
    s0 = _sel(0, _SCALE)
    s1 = _sel(_D, _SCALE)

    def trs(sel, blk):
        return lax.dot_general(
            sel, blk, (((1,), (1,)), ((), ())), preferred_element_type=jnp.float32
        )

    for j in range(_FCB):
        blk = in_ref[pl.ds(j * half, half), :]
        out_ref[j, :, 0:half] = trs(s0, blk)
        out_ref[j, :, half:_ROWS] = trs(s1, blk)


@jax.jit
def _finish(rows128):
    return pl.pallas_call(
        _finish_body,
        grid=(_COLS // _FCB,),
        in_specs=[pl.BlockSpec((_FCB * _ROWS // 2, 128), lambda i: (i, 0))],
        out_specs=pl.BlockSpec((_FCB, _D, _ROWS), lambda i: (i, 0, 0)),
        out_shape=jax.ShapeDtypeStruct((_COLS, _D, _ROWS), jnp.float32),
    )(rows128)


def kernel(x, embedding_weight):
    tab = _pack_table(embedding_weight.T).reshape(2 * _TPACK, _D)
    x_pre = (
        x.astype(jnp.int32)
        .T.reshape(_COLS, 2, _ROWS // 2)
        .transpose(0, 2, 1)
        .reshape(_NW, _NCHUNK, _CHUNK)
    )
    rows = _gather(x_pre, tab)
    out_phys = _finish(rows.reshape(_B // 2, 128))
    return jnp.transpose(out_phys, (2, 0, 1))


# speedup vs baseline: 1.0375x; 1.0375x over previous
import functools
from math import sqrt

import jax
import jax.numpy as jnp
from jax import lax
from jax.experimental import pallas as pl
from jax.experimental.pallas import tpu as pltpu
from jax.experimental.pallas import tpu_sc as plsc

_VOCAB = 1000000
_D = 64
_ROWS = 4096
_COLS = 200
_B = _ROWS * _COLS
_NC = 2
_NS = 16
_NW = _NC * _NS
_PER_W = _B // _NW
_CHUNK = 128
_NCHUNK = _PER_W // _CHUNK
_NBUF = 8
_GDEPTH = 4
_SCALE = sqrt(_D)
_TW = 4096
_TMAIN = (_VOCAB // (2 * _TW)) * (2 * _TW)
_TGRID = _TMAIN // (2 * _TW) + 1
_TTAIL = (_VOCAB - _TMAIN) // 2
_TPACK = _TGRID * _TW


def _eye(scale):
    r = lax.broadcasted_iota(jnp.int32, (_D, _D), 0)
    c = lax.broadcasted_iota(jnp.int32, (_D, _D), 1)
    return jnp.where(r == c, jnp.float32(scale), jnp.float32(0.0))


def _sel(shift, scale):
    r = lax.broadcasted_iota(jnp.int32, (_D, 128), 0)
    c = lax.broadcasted_iota(jnp.int32, (_D, 128), 1)
    return jnp.where(c == r + shift, jnp.float32(scale), jnp.float32(0.0))


def _tr(a, eye):
    return lax.dot_general(
        a, eye, (((0,), (0,)), ((), ())), preferred_element_type=jnp.float32
    )


def _pack_table_body(in0_ref, in1_ref, tail_ref, out_ref):
    i = pl.program_id(0)

    @pl.when(i < _TGRID - 1)
    def _():
        out_ref[...] = _tr(in0_ref[...], _sel(0, 1.0)) + _tr(
            in1_ref[...], _sel(_D, 1.0)
        )

    @pl.when(i == _TGRID - 1)
    def _():
        eye = _eye(1.0)
        out_ref[0:_TTAIL, 0:_D] = _tr(tail_ref[:, 0:_TTAIL], eye)
        out_ref[0:_TTAIL, _D:128] = _tr(tail_ref[:, _TTAIL : 2 * _TTAIL], eye)


@jax.jit
def _pack_table(w_t):
    last = _TGRID - 2

    return pl.pallas_call(
        _pack_table_body,
        grid=(_TGRID,),
        in_specs=[
            pl.BlockSpec((_D, _TW), lambda i: (0, 2 * jnp.minimum(i, last))),
            pl.BlockSpec((_D, _TW), lambda i: (0, 2 * jnp.minimum(i, last) + 1)),
            pl.BlockSpec((_D, 2 * _TTAIL), lambda i: (0, 0)),
        ],
        out_specs=pl.BlockSpec((_TW, 128), lambda i: (i, 0)),
        out_shape=jax.ShapeDtypeStruct((_TPACK, 128), jnp.float32),
    )(w_t, w_t, lax.slice(w_t, (0, _TMAIN), (_D, _VOCAB)))


def _gather_body(x_hbm, tab_hbm, out_hbm, idx_v, rows_v, gsem, wsem):
    cid = lax.axis_index("c")
    sid = lax.axis_index("s")
    wid = sid * _NC + cid
    pltpu.sync_copy(x_hbm.at[wid], idx_v)

    @pl.loop(0, _NCHUNK)
    def _remap(k):
        for v in range(_CHUNK // 16):
            sl = pl.ds(v * 16, 16)
            i = idx_v[k, sl]
            t = lax.bitwise_and(i, 2 * _TW - 1)
            main = (
                (i - t)
                + lax.shift_left(lax.bitwise_and(t, _TW - 1), 1)
                + lax.shift_right_logical(t, _TW.bit_length() - 1)
            )
            u = i - _TMAIN
            h = lax.div(u, _TTAIL)
            tail = _TMAIN + 2 * (u - _TTAIL * h) + h
            idx_v[k, sl] = jnp.where(i < _TMAIN, main, tail)

    def fire_gather(k, b):
        pltpu.async_copy(tab_hbm.at[idx_v.at[k]], rows_v.at[b], gsem)

    def drain_gather(b):
        pltpu.make_async_copy(tab_hbm.at[idx_v.at[0]], rows_v.at[b], gsem).wait()

    def fire_write(k, b):
        pltpu.async_copy(rows_v.at[b], out_hbm.at[wid, k], wsem)

    def drain_write(b):
        pltpu.make_async_copy(rows_v.at[b], out_hbm.at[wid, 0], wsem).wait()

    for b in range(_GDEPTH):
        fire_gather(b, b)

    @pl.loop(0, _NCHUNK - _NBUF, step=_NBUF)
    def _group(j):
        for b in range(_NBUF):
            k = j + b
            drain_gather(b)
            fire_write(k, b)
            b4 = (b + _GDEPTH) % _NBUF
            if b >= _GDEPTH:
                drain_write(b4)
            else:

                @pl.when(j > 0)
                def _():
                    drain_write(b4)

            fire_gather(k + _GDEPTH, b4)

    for b in range(_NBUF):
        k = _NCHUNK - _NBUF + b
        drain_gather(b)
        fire_write(k, b)
        if b < _GDEPTH:
            drain_write((b + _GDEPTH) % _NBUF)
            fire_gather(k + _GDEPTH, (b + _GDEPTH) % _NBUF)
    for b in range(_NBUF):
        drain_write(b)


@jax.jit
def _gather(x_flat, tab):
    k = pl.kernel(
        _gather_body,
        out_type=jax.ShapeDtypeStruct((_NW, _NCHUNK, _CHUNK, _D), jnp.float32),
        mesh=plsc.VectorSubcoreMesh(core_axis_name="c", subcore_axis_name="s"),
        scratch_types=[
            pltpu.VMEM((_NCHUNK, _CHUNK), jnp.int32),
            pltpu.VMEM((_NBUF, _CHUNK, _D), jnp.float32),
            pltpu.SemaphoreType.DMA,
            pltpu.SemaphoreType.DMA,
        ],
        compiler_params=pltpu.CompilerParams(use_tc_tiling_on_sc=False),
    )
    return k(x_flat, tab)


_FCB = 4


def _finish_body(in_ref, out_ref):
    half = _ROWS // 2
    s0 = _sel(0, _SCALE)
    s1 = _sel(_D, _SCALE)

    def trs(sel, blk):
        return lax.dot_general(
            sel, blk, (((1,), (1,)), ((), ())), preferred_element_type=jnp.float32
        )

    for j in range(_FCB):
        blk = in_ref[pl.ds(j * half, half), :]
        out_ref[j, :, 0:half] = trs(s0, blk)
        out_ref[j, :, half:_ROWS] = trs(s1, blk)


@jax.jit
def _finish(rows128):
    return pl.pallas_call(
        _finish_body,
        grid=(_COLS // _FCB,),
        in_specs=[pl.BlockSpec((_FCB * _ROWS // 2, 128), lambda i: (i, 0))],
        out_specs=pl.BlockSpec((_FCB, _D, _ROWS), lambda i: (i, 0, 0)),
        out_shape=jax.ShapeDtypeStruct((_COLS, _D, _ROWS), jnp.float32),
    )(rows128)


def kernel(x, embedding_weight):
    tab = _pack_table(embedding_weight.T).reshape(2 * _TPACK, _D)
    x_pre = (
        x.astype(jnp.int32)
        .T.reshape(_COLS, 2, _ROWS // 2)
        .transpose(0, 2, 1)
        .reshape(_NW, _NCHUNK, _CHUNK)
    )
    rows = _gather(x_pre, tab)
    out_phys = _finish(rows.reshape(_B // 2, 128))
    return jnp.transpose(out_phys, (2, 0, 1))
